# Initial kernel scaffold; baseline (speedup 1.0000x reference)
import jax
import jax.numpy as jnp
from jax.experimental import pallas as pl


def kernel(z, W):
    raise NotImplementedError("write your pallas kernel here")


# speedup vs baseline: 1.3040x; 1.3040x over previous
import functools

import jax
import jax.numpy as jnp
from jax import lax
from jax.experimental import pallas as pl
from jax.experimental.pallas import tpu as pltpu
from jax.experimental.pallas import tpu_sc as plsc

_COMMITMENT_COST = 0.25
_BM = 512
_NTILE = 4096

_SC_CORES = 2
_SC_SUBCORES = 16
_NW = _SC_CORES * _SC_SUBCORES
_ICH = 128


def _dist_argmin_body(zb_ref, z2_ref, w_ref, w2_ref, idx_ref):
    n_total = w_ref.shape[0]
    run_min = None
    run_idx = None
    for c in range(0, n_total, _NTILE):
        wblk = w_ref[c:c + _NTILE, :]
        w2 = w2_ref[:, c:c + _NTILE]
        mm = lax.dot_general(zb_ref[...], wblk, (((1,), (1,)), ((), ())),
                             preferred_element_type=jnp.float32)
        d = (z2_ref[...] - mm) + w2
        mn = jnp.min(d, axis=1, keepdims=True)
        col = lax.broadcasted_iota(jnp.int32, d.shape, 1) + c
        cidx = jnp.min(jnp.where(d == mn, col, jnp.int32(2 ** 30)),
                       axis=1, keepdims=True)
        if run_min is None:
            run_idx = cidx
        else:
            upd = mn < run_min
            run_idx = jnp.where(upd, cidx, run_idx)
            mn = jnp.where(upd, mn, run_min)
        run_min = mn.astype(jnp.bfloat16).astype(jnp.float32)
    idx_ref[...] = run_idx


def _tc_dist_argmin(zb, z2, W, w2):
    m, d = zb.shape
    n = W.shape[0]
    grid = (m // _BM,)
    return pl.pallas_call(
        _dist_argmin_body,
        grid=grid,
        in_specs=[
            pl.BlockSpec((_BM, d), lambda i: (i, 0)),
            pl.BlockSpec((_BM, 1), lambda i: (i, 0)),
            pl.BlockSpec((n, d), lambda i: (0, 0)),
            pl.BlockSpec((1, n), lambda i: (0, 0)),
        ],
        out_specs=pl.BlockSpec((_BM, 1), lambda i: (i, 0)),
        out_shape=jax.ShapeDtypeStruct((m, 1), jnp.int32),
    )(zb, z2, W, w2)


def _sc_gather(table, idx_flat):
    m = idx_flat.shape[0]
    d = table.shape[1]
    rpw = m // _NW
    nch = rpw // _ICH
    idx3 = idx_flat.reshape(_NW, nch, _ICH)
    mesh = plsc.VectorSubcoreMesh(core_axis_name="c", subcore_axis_name="s")

    @functools.partial(
        pl.kernel, mesh=mesh,
        out_type=jax.ShapeDtypeStruct((m, d), jnp.float32),
        compiler_params=pltpu.CompilerParams(use_tc_tiling_on_sc=False),
        scratch_types=[
            pltpu.VMEM((nch, _ICH), jnp.int32),
            pltpu.VMEM((rpw, d), jnp.float32),
            pltpu.SemaphoreType.DMA,
        ],
    )
    def gk(table_hbm, idx_hbm, out_hbm, idx_v, rows_v, sem):
        wid = lax.axis_index("s") * _SC_CORES + lax.axis_index("c")
        pltpu.sync_copy(idx_hbm.at[wid], idx_v)
        for cc in range(nch):
            pltpu.async_copy(table_hbm.at[idx_v.at[cc]],
                             rows_v.at[pl.ds(cc * _ICH, _ICH)], sem).wait()
        pltpu.sync_copy(rows_v, out_hbm.at[pl.ds(wid * rpw, rpw)])

    return gk(table, idx3)


def kernel(z, W):
    B, T, D = z.shape
    N = W.shape[0]
    flat_z = z.reshape(-1, D)
    zb = (2.0 * flat_z).astype(jnp.bfloat16)
    z2 = jnp.sum(flat_z ** 2, axis=-1, keepdims=True)
    w2 = jnp.sum(W ** 2, axis=-1).reshape(1, N)
    idx2d = _tc_dist_argmin(zb, z2, W, w2)
    indices_flat = idx2d.reshape(-1)
    z_q_flat = _sc_gather(W, indices_flat)
    z_q = z_q_flat.reshape(B, T, D)
    loss = _COMMITMENT_COST * jnp.mean((lax.stop_gradient(z_q) - z) ** 2)
    z_q_st = z + lax.stop_gradient(z_q - z)
    indices = indices_flat.reshape(B, T)
    return (z_q_st, loss, indices)


# speedup vs baseline: 1.4228x; 1.0911x over previous
import functools

import jax
import jax.numpy as jnp
from jax import lax
from jax.experimental import pallas as pl
from jax.experimental.pallas import tpu as pltpu
from jax.experimental.pallas import tpu_sc as plsc

_COMMITMENT_COST = 0.25
_BM = 512
_NTILE = 4096

_SC_CORES = 2
_SC_SUBCORES = 16
_NW = _SC_CORES * _SC_SUBCORES
_ICH = 128


def _dist_argmin_body(zb_ref, z2_ref, w_ref, w2_ref, iota_ref, idx_ref):
    n_total = w_ref.shape[0]
    run_min = None
    run_idx = None
    for c in range(0, n_total, _NTILE):
        wblk = w_ref[c:c + _NTILE, :]
        w2 = w2_ref[:, c:c + _NTILE]
        mm = lax.dot_general(zb_ref[...], wblk, (((1,), (1,)), ((), ())),
                             preferred_element_type=jnp.float32)
        d = (z2_ref[...] - mm) + w2
        mn = jnp.min(d, axis=1, keepdims=True)
        col = iota_ref[:, c:c + _NTILE]
        cidx = jnp.min(jnp.where(d == mn, col, jnp.float32(jnp.inf)),
                       axis=1, keepdims=True)
        if run_min is None:
            run_idx = cidx
        else:
            upd = mn < run_min
            run_idx = jnp.where(upd, cidx, run_idx)
            mn = jnp.where(upd, mn, run_min)
        run_min = mn.astype(jnp.bfloat16).astype(jnp.float32)
    idx_ref[...] = run_idx.astype(jnp.int32)


def _tc_dist_argmin(zb, z2, W, w2, iota_row):
    m, d = zb.shape
    n = W.shape[0]
    grid = (m // _BM,)
    return pl.pallas_call(
        _dist_argmin_body,
        grid=grid,
        in_specs=[
            pl.BlockSpec((_BM, d), lambda i: (i, 0)),
            pl.BlockSpec((_BM, 1), lambda i: (i, 0)),
            pl.BlockSpec((n, d), lambda i: (0, 0)),
            pl.BlockSpec((1, n), lambda i: (0, 0)),
            pl.BlockSpec((1, n), lambda i: (0, 0)),
        ],
        out_specs=pl.BlockSpec((_BM, 1), lambda i: (i, 0)),
        out_shape=jax.ShapeDtypeStruct((m, 1), jnp.int32),
    )(zb, z2, W, w2, iota_row)


def _sc_gather(table, idx_flat):
    m = idx_flat.shape[0]
    d = table.shape[1]
    rpw = m // _NW
    nch = rpw // _ICH
    idx3 = idx_flat.reshape(_NW, nch, _ICH)
    mesh = plsc.VectorSubcoreMesh(core_axis_name="c", subcore_axis_name="s")

    @functools.partial(
        pl.kernel, mesh=mesh,
        out_type=jax.ShapeDtypeStruct((m, d), jnp.float32),
        compiler_params=pltpu.CompilerParams(use_tc_tiling_on_sc=False),
        scratch_types=[
            pltpu.VMEM((nch, _ICH), jnp.int32),
            pltpu.VMEM((rpw, d), jnp.float32),
            pltpu.SemaphoreType.DMA,
        ],
    )
    def gk(table_hbm, idx_hbm, out_hbm, idx_v, rows_v, sem):
        wid = lax.axis_index("s") * _SC_CORES + lax.axis_index("c")
        pltpu.sync_copy(idx_hbm.at[wid], idx_v)
        for cc in range(nch):
            pltpu.async_copy(table_hbm.at[idx_v.at[cc]],
                             rows_v.at[pl.ds(cc * _ICH, _ICH)], sem).wait()
        pltpu.sync_copy(rows_v, out_hbm.at[pl.ds(wid * rpw, rpw)])

    return gk(table, idx3)


def kernel(z, W):
    B, T, D = z.shape
    N = W.shape[0]
    flat_z = z.reshape(-1, D)
    zb = (2.0 * flat_z).astype(jnp.bfloat16)
    z2 = jnp.sum(flat_z ** 2, axis=-1, keepdims=True)
    w2 = jnp.sum(W ** 2, axis=-1).reshape(1, N)
    iota_row = jnp.arange(N, dtype=jnp.float32).reshape(1, N)
    idx2d = _tc_dist_argmin(zb, z2, W, w2, iota_row)
    indices_flat = idx2d.reshape(-1)
    z_q_flat = _sc_gather(W, indices_flat)
    z_q = z_q_flat.reshape(B, T, D)
    loss = _COMMITMENT_COST * jnp.mean((lax.stop_gradient(z_q) - z) ** 2)
    z_q_st = z + lax.stop_gradient(z_q - z)
    indices = indices_flat.reshape(B, T)
    return (z_q_st, loss, indices)


# speedup vs baseline: 1.4804x; 1.0406x over previous
import functools

import jax
import jax.numpy as jnp
from jax import lax
from jax.experimental import pallas as pl
from jax.experimental.pallas import tpu as pltpu
from jax.experimental.pallas import tpu_sc as plsc

_COMMITMENT_COST = 0.25
_BM = 1024
_NTILE = 4096

_SC_CORES = 2
_SC_SUBCORES = 16
_NW = _SC_CORES * _SC_SUBCORES
_ICH = 128


def _dist_argmin_body(z_ref, w_ref, w2_ref, iota_ref, idx_ref):
    n_total = w_ref.shape[0]
    zf = z_ref[...]
    zb = (2.0 * zf).astype(jnp.bfloat16)
    z2 = jnp.sum(zf ** 2, axis=1, keepdims=True)
    run_min = None
    run_idx = None
    for c in range(0, n_total, _NTILE):
        wblk = w_ref[c:c + _NTILE, :]
        w2 = w2_ref[:, c:c + _NTILE]
        mm = lax.dot_general(zb, wblk, (((1,), (1,)), ((), ())),
                             preferred_element_type=jnp.float32)
        d = (z2 - mm) + w2
        mn = jnp.min(d, axis=1, keepdims=True)
        col = iota_ref[:, c:c + _NTILE]
        cidx = jnp.min(jnp.where(d == mn, col, jnp.float32(jnp.inf)),
                       axis=1, keepdims=True)
        if run_min is None:
            run_idx = cidx
        else:
            upd = mn < run_min
            run_idx = jnp.where(upd, cidx, run_idx)
            mn = jnp.where(upd, mn, run_min)
        run_min = mn.astype(jnp.bfloat16).astype(jnp.float32)
    idx_ref[...] = run_idx.astype(jnp.int32)


def _tc_dist_argmin(flat_z, W, w2, iota_row):
    m, d = flat_z.shape
    n = W.shape[0]
    grid = (m // _BM,)
    return pl.pallas_call(
        _dist_argmin_body,
        grid=grid,
        in_specs=[
            pl.BlockSpec((_BM, d), lambda i: (i, 0)),
            pl.BlockSpec((n, d), lambda i: (0, 0)),
            pl.BlockSpec((1, n), lambda i: (0, 0)),
            pl.BlockSpec((1, n), lambda i: (0, 0)),
        ],
        out_specs=pl.BlockSpec((_BM, 1), lambda i: (i, 0)),
        out_shape=jax.ShapeDtypeStruct((m, 1), jnp.int32),
    )(flat_z, W, w2, iota_row)


def _sc_gather(table, idx_flat):
    m = idx_flat.shape[0]
    d = table.shape[1]
    rpw = m // _NW
    nch = rpw // _ICH
    idx3 = idx_flat.reshape(_NW, nch, _ICH)
    mesh = plsc.VectorSubcoreMesh(core_axis_name="c", subcore_axis_name="s")

    @functools.partial(
        pl.kernel, mesh=mesh,
        out_type=jax.ShapeDtypeStruct((m, d), jnp.float32),
        compiler_params=pltpu.CompilerParams(use_tc_tiling_on_sc=False),
        scratch_types=[
            pltpu.VMEM((nch, _ICH), jnp.int32),
            pltpu.VMEM((rpw, d), jnp.float32),
            pltpu.SemaphoreType.DMA,
        ],
    )
    def gk(table_hbm, idx_hbm, out_hbm, idx_v, rows_v, sem):
        wid = lax.axis_index("s") * _SC_CORES + lax.axis_index("c")
        pltpu.sync_copy(idx_hbm.at[wid], idx_v)
        for cc in range(nch):
            pltpu.async_copy(table_hbm.at[idx_v.at[cc]],
                             rows_v.at[pl.ds(cc * _ICH, _ICH)], sem).wait()
        pltpu.sync_copy(rows_v, out_hbm.at[pl.ds(wid * rpw, rpw)])

    return gk(table, idx3)


def kernel(z, W):
    B, T, D = z.shape
    N = W.shape[0]
    flat_z = z.reshape(-1, D)
    w2 = jnp.sum(W ** 2, axis=-1).reshape(1, N)
    iota_row = jnp.arange(N, dtype=jnp.float32).reshape(1, N)
    idx2d = _tc_dist_argmin(flat_z, W, w2, iota_row)
    indices_flat = idx2d.reshape(-1)
    z_q_flat = _sc_gather(W, indices_flat)
    z_q = z_q_flat.reshape(B, T, D)
    loss = _COMMITMENT_COST * jnp.mean((lax.stop_gradient(z_q) - z) ** 2)
    z_q_st = z + lax.stop_gradient(z_q - z)
    indices = indices_flat.reshape(B, T)
    return (z_q_st, loss, indices)


# speedup vs baseline: 1.5035x; 1.0156x over previous
import functools

import jax
import jax.numpy as jnp
from jax import lax
from jax.experimental import pallas as pl
from jax.experimental.pallas import tpu as pltpu
from jax.experimental.pallas import tpu_sc as plsc

_COMMITMENT_COST = 0.25
_BM = 2048
_NTILE = 4096

_SC_CORES = 2
_SC_SUBCORES = 16
_NW = _SC_CORES * _SC_SUBCORES
_ICH = 128


def _dist_argmin_body(z_ref, w_ref, w2_ref, iota_ref, idx_ref):
    n_total = w_ref.shape[0]
    zf = z_ref[...]
    zb = (2.0 * zf).astype(jnp.bfloat16)
    z2 = jnp.sum(zf ** 2, axis=1, keepdims=True)
    run_min = None
    run_idx = None
    for c in range(0, n_total, _NTILE):
        wblk = w_ref[c:c + _NTILE, :]
        w2 = w2_ref[:, c:c + _NTILE]
        mm = lax.dot_general(zb, wblk, (((1,), (1,)), ((), ())),
                             preferred_element_type=jnp.float32)
        d = (z2 - mm) + w2
        mn = jnp.min(d, axis=1, keepdims=True)
        col = iota_ref[:, c:c + _NTILE]
        cidx = jnp.min(jnp.where(d == mn, col, jnp.float32(jnp.inf)),
                       axis=1, keepdims=True)
        if run_min is None:
            run_idx = cidx
        else:
            upd = mn < run_min
            run_idx = jnp.where(upd, cidx, run_idx)
            mn = jnp.where(upd, mn, run_min)
        run_min = mn.astype(jnp.bfloat16).astype(jnp.float32)
    idx_ref[...] = run_idx.astype(jnp.int32)


def _tc_dist_argmin(flat_z, W, w2, iota_row):
    m, d = flat_z.shape
    n = W.shape[0]
    grid = (m // _BM,)
    return pl.pallas_call(
        _dist_argmin_body,
        grid=grid,
        in_specs=[
            pl.BlockSpec((_BM, d), lambda i: (i, 0)),
            pl.BlockSpec((n, d), lambda i: (0, 0)),
            pl.BlockSpec((1, n), lambda i: (0, 0)),
            pl.BlockSpec((1, n), lambda i: (0, 0)),
        ],
        out_specs=pl.BlockSpec((_BM, 1), lambda i: (i, 0)),
        out_shape=jax.ShapeDtypeStruct((m, 1), jnp.int32),
    )(flat_z, W, w2, iota_row)


def _sc_gather(table, idx_flat):
    m = idx_flat.shape[0]
    d = table.shape[1]
    rpw = m // _NW
    nch = rpw // _ICH
    idx3 = idx_flat.reshape(_NW, nch, _ICH)
    mesh = plsc.VectorSubcoreMesh(core_axis_name="c", subcore_axis_name="s")

    @functools.partial(
        pl.kernel, mesh=mesh,
        out_type=jax.ShapeDtypeStruct((m, d), jnp.float32),
        compiler_params=pltpu.CompilerParams(use_tc_tiling_on_sc=False),
        scratch_types=[
            pltpu.VMEM((nch, _ICH), jnp.int32),
            pltpu.VMEM((rpw, d), jnp.float32),
            pltpu.SemaphoreType.DMA,
        ],
    )
    def gk(table_hbm, idx_hbm, out_hbm, idx_v, rows_v, sem):
        wid = lax.axis_index("s") * _SC_CORES + lax.axis_index("c")
        pltpu.sync_copy(idx_hbm.at[wid], idx_v)
        for cc in range(nch):
            pltpu.async_copy(table_hbm.at[idx_v.at[cc]],
                             rows_v.at[pl.ds(cc * _ICH, _ICH)], sem).wait()
        pltpu.sync_copy(rows_v, out_hbm.at[pl.ds(wid * rpw, rpw)])

    return gk(table, idx3)


def kernel(z, W):
    B, T, D = z.shape
    N = W.shape[0]
    flat_z = z.reshape(-1, D)
    w2 = jnp.sum(W ** 2, axis=-1).reshape(1, N)
    iota_row = jnp.arange(N, dtype=jnp.float32).reshape(1, N)
    idx2d = _tc_dist_argmin(flat_z, W, w2, iota_row)
    indices_flat = idx2d.reshape(-1)
    z_q_flat = _sc_gather(W, indices_flat)
    z_q = z_q_flat.reshape(B, T, D)
    loss = _COMMITMENT_COST * jnp.mean((lax.stop_gradient(z_q) - z) ** 2)
    z_q_st = z + lax.stop_gradient(z_q - z)
    indices = indices_flat.reshape(B, T)
    return (z_q_st, loss, indices)
